# Initial kernel scaffold; baseline (speedup 1.0000x reference)
import jax
import jax.numpy as jnp
from jax.experimental import pallas as pl


def kernel(x, edge_index, batch, W1_0, b1_0, g_0, be_0, W2_0, b2_0, W1_1, b1_1, g_1, be_1, W2_1, b2_1, W1_2, b1_2, g_2, be_2, W2_2, b2_2, Wf1, bf1, Wf2, bf2):
    raise NotImplementedError("write your pallas kernel here")


# speedup vs baseline: 3.2034x; 3.2034x over previous
import functools

import jax
import jax.numpy as jnp
from jax import lax
from jax.experimental import pallas as pl
from jax.experimental.pallas import tpu as pltpu
from jax.experimental.pallas import tpu_sc as plsc

_LANE = 128
_NT = 16
_NC = 2
_CHUNK = 128


def _agg(x_cb, srct, dstt, n_pad):
    ncb, n, lane = x_cb.shape
    nch = srct.shape[1]
    npass = ncb // _NC
    rpt = (n // _NT) & ~7
    tail0 = rpt * _NT
    tail = n - tail0
    mesh = plsc.VectorSubcoreMesh(core_axis_name="c", subcore_axis_name="s")

    @functools.partial(
        pl.kernel,
        mesh=mesh,
        out_type=jax.ShapeDtypeStruct((ncb, n, lane), jnp.float32),
        scratch_types=[
            pltpu.VMEM((nch, _CHUNK), jnp.int32),
            pltpu.VMEM((nch, _CHUNK), jnp.int32),
            pltpu.VMEM((_CHUNK, lane), jnp.float32),
            pltpu.VMEM_SHARED((n_pad, lane), jnp.float32),
            pltpu.SemaphoreType.DMA,
        ],
    )
    def body(x_hbm, srct_hbm, dstt_hbm, out_hbm, src_v, dst_v, rows_v, acc_sh, sem):
        cid = lax.axis_index("c")
        sid = lax.axis_index("s")
        pltpu.sync_copy(srct_hbm.at[sid], src_v)
        pltpu.sync_copy(dstt_hbm.at[sid], dst_v)
        r0 = sid * rpt
        for p in range(npass):
            blk = cid * npass + p
            pltpu.sync_copy(x_hbm.at[blk, pl.ds(r0, rpt)], acc_sh.at[pl.ds(r0, rpt)])
            if tail:
                @pl.when(sid == 0)
                def _():
                    pltpu.sync_copy(x_hbm.at[blk, pl.ds(tail0, tail)],
                                    acc_sh.at[pl.ds(tail0, tail)])
            plsc.subcore_barrier()

            def chunk(j, carry):
                pltpu.async_copy(x_hbm.at[blk].at[src_v.at[j]], rows_v, sem).wait()
                pltpu.sync_copy(rows_v, acc_sh.at[dst_v.at[j]], add=True)
                return carry

            lax.fori_loop(0, nch, chunk, 0)
            plsc.subcore_barrier()
            pltpu.sync_copy(acc_sh.at[pl.ds(r0, rpt)], out_hbm.at[blk, pl.ds(r0, rpt)])
            if tail:
                @pl.when(sid == 0)
                def _():
                    pltpu.sync_copy(acc_sh.at[pl.ds(tail0, tail)],
                                    out_hbm.at[blk, pl.ds(tail0, tail)])
            if p + 1 < npass:
                plsc.subcore_barrier()

    return body(x_cb, srct, dstt)


def _mlp_layer(h_cb, W1b, b1, g, be, W2, b2, block_rows):
    ncb_in, n, _ = h_cb.shape
    H = W2.shape[0]
    ncb_out = H // _LANE
    nb = n // block_rows

    def body(h_ref, W1_ref, b1_ref, g_ref, be_ref, W2_ref, b2_ref, out_ref,
             sum_ref, sq_ref):
        p = pl.program_id(0)
        i = pl.program_id(1)
        h1 = b1_ref[...]
        for c in range(ncb_in):
            h1 = h1 + jnp.dot(h_ref[c], W1_ref[c],
                              preferred_element_type=jnp.float32)

        @pl.when(jnp.logical_and(p == 0, i == 0))
        def _():
            sum_ref[...] = jnp.zeros_like(sum_ref)
            sq_ref[...] = jnp.zeros_like(sq_ref)

        @pl.when(p == 0)
        def _():
            sum_ref[...] += jnp.sum(h1, axis=0, keepdims=True)
            sq_ref[...] += jnp.sum(h1 * h1, axis=0, keepdims=True)

        @pl.when(p == 1)
        def _():
            inv_n = 1.0 / n
            mu = sum_ref[...] * inv_n
            var = sq_ref[...] * inv_n - mu * mu
            scale = g_ref[...] * lax.rsqrt(var + 1e-5)
            shift = be_ref[...] - mu * scale
            r = jnp.maximum(h1 * scale + shift, 0.0)
            h2 = jnp.dot(r, W2_ref[...], preferred_element_type=jnp.float32)
            y = jnp.maximum(h2 + b2_ref[...], 0.0)
            for c in range(ncb_out):
                out_ref[c] = y[:, c * _LANE:(c + 1) * _LANE]

    return pl.pallas_call(
        body,
        grid=(2, nb),
        in_specs=[
            pl.BlockSpec((ncb_in, block_rows, _LANE), lambda p, i: (0, i, 0)),
            pl.BlockSpec((ncb_in, _LANE, H), lambda p, i: (0, 0, 0)),
            pl.BlockSpec((1, H), lambda p, i: (0, 0)),
            pl.BlockSpec((1, H), lambda p, i: (0, 0)),
            pl.BlockSpec((1, H), lambda p, i: (0, 0)),
            pl.BlockSpec((H, H), lambda p, i: (0, 0)),
            pl.BlockSpec((1, H), lambda p, i: (0, 0)),
        ],
        out_specs=pl.BlockSpec((ncb_out, block_rows, _LANE), lambda p, i: (0, i, 0)),
        out_shape=jax.ShapeDtypeStruct((ncb_out, n, _LANE), jnp.float32),
        scratch_shapes=[
            pltpu.VMEM((1, H), jnp.float32),
            pltpu.VMEM((1, H), jnp.float32),
        ],
    )(h_cb, W1b, b1, g, be, W2, b2)


def _mlp_final(h_cb, batch_r, W1b, b1, g, be, W2, b2, Wf1, bf1, Wf2, bf2,
               n_graphs, block_rows):
    ncb_in, n, _ = h_cb.shape
    H = W2.shape[0]
    OUT = Wf2.shape[1]
    nb = n // block_rows

    def body(h_ref, batch_ref, W1_ref, b1_ref, g_ref, be_ref, W2_ref, b2_ref,
             Wf1_ref, bf1_ref, Wf2_ref, bf2_ref, out_ref, sum_ref, sq_ref,
             pool_ref):
        p = pl.program_id(0)
        i = pl.program_id(1)
        h1 = b1_ref[...]
        for c in range(ncb_in):
            h1 = h1 + jnp.dot(h_ref[c], W1_ref[c],
                              preferred_element_type=jnp.float32)

        @pl.when(jnp.logical_and(p == 0, i == 0))
        def _():
            sum_ref[...] = jnp.zeros_like(sum_ref)
            sq_ref[...] = jnp.zeros_like(sq_ref)
            pool_ref[...] = jnp.zeros_like(pool_ref)

        @pl.when(p == 0)
        def _():
            sum_ref[...] += jnp.sum(h1, axis=0, keepdims=True)
            sq_ref[...] += jnp.sum(h1 * h1, axis=0, keepdims=True)

        @pl.when(p == 1)
        def _():
            inv_n = 1.0 / n
            mu = sum_ref[...] * inv_n
            var = sq_ref[...] * inv_n - mu * mu
            scale = g_ref[...] * lax.rsqrt(var + 1e-5)
            shift = be_ref[...] - mu * scale
            r = jnp.maximum(h1 * scale + shift, 0.0)
            h2 = jnp.dot(r, W2_ref[...], preferred_element_type=jnp.float32)
            y = jnp.maximum(h2 + b2_ref[...], 0.0)
            ids = batch_ref[0, 0, :]
            oh = (ids[:, None] == lax.broadcasted_iota(
                jnp.int32, (1, n_graphs), 1)).astype(jnp.float32)
            pool_ref[...] += lax.dot_general(
                oh, y, (((0,), (0,)), ((), ())),
                preferred_element_type=jnp.float32)

        @pl.when(jnp.logical_and(p == 1, i == nb - 1))
        def _():
            pooled = pool_ref[...]
            hf = jnp.maximum(
                jnp.dot(pooled, Wf1_ref[...], preferred_element_type=jnp.float32)
                + bf1_ref[...], 0.0)
            out_ref[...] = (jnp.dot(hf, Wf2_ref[...],
                                    preferred_element_type=jnp.float32)
                            + bf2_ref[...])

    return pl.pallas_call(
        body,
        grid=(2, nb),
        in_specs=[
            pl.BlockSpec((ncb_in, block_rows, _LANE), lambda p, i: (0, i, 0)),
            pl.BlockSpec((1, 1, block_rows), lambda p, i: (i, 0, 0)),
            pl.BlockSpec((ncb_in, _LANE, H), lambda p, i: (0, 0, 0)),
            pl.BlockSpec((1, H), lambda p, i: (0, 0)),
            pl.BlockSpec((1, H), lambda p, i: (0, 0)),
            pl.BlockSpec((1, H), lambda p, i: (0, 0)),
            pl.BlockSpec((H, H), lambda p, i: (0, 0)),
            pl.BlockSpec((1, H), lambda p, i: (0, 0)),
            pl.BlockSpec((H, H), lambda p, i: (0, 0)),
            pl.BlockSpec((1, H), lambda p, i: (0, 0)),
            pl.BlockSpec((H, OUT), lambda p, i: (0, 0)),
            pl.BlockSpec((1, OUT), lambda p, i: (0, 0)),
        ],
        out_specs=pl.BlockSpec((n_graphs, OUT), lambda p, i: (0, 0)),
        out_shape=jax.ShapeDtypeStruct((n_graphs, OUT), jnp.float32),
        scratch_shapes=[
            pltpu.VMEM((1, H), jnp.float32),
            pltpu.VMEM((1, H), jnp.float32),
            pltpu.VMEM((n_graphs, H), jnp.float32),
        ],
    )(h_cb, batch_r, W1b, b1, g, be, W2, b2, Wf1, bf1, Wf2, bf2)


def kernel(x, edge_index, batch,
           W1_0, b1_0, g_0, be_0, W2_0, b2_0,
           W1_1, b1_1, g_1, be_1, W2_1, b2_1,
           W1_2, b1_2, g_2, be_2, W2_2, b2_2,
           Wf1, bf1, Wf2, bf2):
    n, din = x.shape
    e = edge_index.shape[1]
    H = W2_0.shape[0]
    G = 128
    n_pad = n + 8
    block_rows = 1000

    x_cb = jnp.transpose(x.reshape(n, din // _LANE, _LANE), (1, 0, 2))
    nch = (-(-e // _NT) + _CHUNK - 1) // _CHUNK
    e_pad = _NT * nch * _CHUNK
    src = jnp.concatenate(
        [edge_index[0], jnp.zeros((e_pad - e,), dtype=jnp.int32)])
    dst = jnp.concatenate(
        [edge_index[1], jnp.full((e_pad - e,), n, dtype=jnp.int32)])
    srct = src.reshape(_NT, nch, _CHUNK)
    dstt = dst.reshape(_NT, nch, _CHUNK)
    batch_r = batch.reshape(n // block_rows, 1, block_rows)

    def wblocks(W):
        return W.reshape(W.shape[0] // _LANE, _LANE, W.shape[1])

    def rowvec(v):
        return v.reshape(1, v.shape[0])

    h = _agg(x_cb, srct, dstt, n_pad)
    x1 = _mlp_layer(h, wblocks(W1_0), rowvec(b1_0), rowvec(g_0), rowvec(be_0),
                    W2_0, rowvec(b2_0), block_rows)
    h = _agg(x1, srct, dstt, n_pad)
    x2 = _mlp_layer(h, wblocks(W1_1), rowvec(b1_1), rowvec(g_1), rowvec(be_1),
                    W2_1, rowvec(b2_1), block_rows)
    h = _agg(x2, srct, dstt, n_pad)
    out = _mlp_final(h, batch_r, wblocks(W1_2), rowvec(b1_2), rowvec(g_2),
                     rowvec(be_2), W2_2, rowvec(b2_2), Wf1, rowvec(bf1),
                     Wf2, rowvec(bf2), G, block_rows)
    return out
